# baseline (speedup 1.0000x reference)
import functools

import jax
import jax.numpy as jnp
from jax import lax
from jax.experimental import pallas as pl
from jax.experimental.pallas import tpu as pltpu
from jax.experimental.pallas import tpu_sc as plsc

_B = 16384
_F = 26
_V = 100000
_D = 16
_NC = 13
_H1 = 512
_H2 = 256
_EPS = 1e-5

_BF = _B * _F
_NW = 32
_PER_W = _BF // _NW
_NCHUNK = 4
_CHUNK = _PER_W // _NCHUNK

_BB = 2048
_NBLK = _B // _BB


def _gather_body(tbl_hbm, idx_hbm, out_hbm, idx_v, rows_v, sem):
    c = lax.axis_index("c")
    s = lax.axis_index("s")
    wid = s * 2 + c
    base = wid * _PER_W
    pltpu.sync_copy(idx_hbm.at[pl.ds(base, _PER_W)], idx_v)
    for k in range(_NCHUNK):
        pltpu.async_copy(
            tbl_hbm.at[0, pl.ds(k * _CHUNK, _CHUNK)], rows_v, sem
        ).wait()
        pltpu.sync_copy(rows_v, out_hbm.at[pl.ds(base + k * _CHUNK, _CHUNK)])


def _sc_gather(tbl_flat, idx):
    mesh = plsc.VectorSubcoreMesh(core_axis_name="c", subcore_axis_name="s")
    f = pl.kernel(
        _gather_body,
        out_type=jax.ShapeDtypeStruct((_BF, _D), jnp.float32),
        mesh=mesh,
        scratch_types=[
            pltpu.VMEM((_PER_W,), jnp.int32),
            pltpu.VMEM((_CHUNK, _D), jnp.float32),
            pltpu.SemaphoreType.DMA,
        ],
        compiler_params=pltpu.CompilerParams(use_tc_tiling_on_sc=False),
    )
    return f(tbl_flat, idx)


def _k1_body(xc_ref, emb_ref, w1e_ref, w1c_ref, b1_ref, gc_ref, bc_ref,
             a1_ref, sq_ref, acc_ref, xst_ref):
    pid = pl.program_id(0)

    @pl.when(pid == 0)
    def _():
        xc = xc_ref[...]
        m = jnp.mean(xc, axis=0)
        v = jnp.mean(xc * xc, axis=0) - m * m
        sc = gc_ref[0, :] * lax.rsqrt(v + _EPS)
        xst_ref[0, :] = sc
        xst_ref[1, :] = bc_ref[0, :] - m * sc
        acc_ref[...] = jnp.zeros_like(acc_ref)

    xcn = xc_ref[pl.ds(pid * _BB, _BB), :] * xst_ref[0, :] + xst_ref[1, :]
    z = (jnp.dot(emb_ref[...], w1e_ref[...], preferred_element_type=jnp.float32)
         + jnp.dot(xcn, w1c_ref[...], preferred_element_type=jnp.float32)
         + b1_ref[0, :])
    a1 = jnp.maximum(z, 0.0)
    a1_ref[...] = a1
    acc_ref[0, :] += jnp.sum(a1, axis=0)
    acc_ref[1, :] += jnp.sum(a1 * a1, axis=0)

    @pl.when(pid == _NBLK - 1)
    def _():
        sq_ref[...] = acc_ref[...]


def _k2_body(a1_ref, sq1_ref, g1_ref, bt1_ref, w2_ref, b2_ref,
             a2_ref, sq2_ref, acc_ref):
    pid = pl.program_id(0)

    @pl.when(pid == 0)
    def _():
        acc_ref[...] = jnp.zeros_like(acc_ref)

    m = sq1_ref[0, :] * (1.0 / _B)
    v = sq1_ref[1, :] * (1.0 / _B) - m * m
    alpha = g1_ref[0, :] * lax.rsqrt(v + _EPS)
    beta = bt1_ref[0, :] - m * alpha
    h1 = a1_ref[...] * alpha + beta
    z = jnp.dot(h1, w2_ref[...], preferred_element_type=jnp.float32) + b2_ref[0, :]
    a2 = jnp.maximum(z, 0.0)
    a2_ref[...] = a2
    acc_ref[0, :] += jnp.sum(a2, axis=0)
    acc_ref[1, :] += jnp.sum(a2 * a2, axis=0)

    @pl.when(pid == _NBLK - 1)
    def _():
        sq2_ref[...] = acc_ref[...]


def _k3_body(a2_ref, sq2_ref, g2_ref, bt2_ref, w3_ref, b3_ref, out_ref):
    m = sq2_ref[0, :] * (1.0 / _B)
    v = sq2_ref[1, :] * (1.0 / _B) - m * m
    alpha = g2_ref[0, :] * lax.rsqrt(v + _EPS)
    beta = bt2_ref[0, :] - m * alpha
    h2 = a2_ref[...] * alpha + beta
    out_ref[...] = (jnp.dot(h2, w3_ref[...], preferred_element_type=jnp.float32)
                    + b3_ref[0, :])


def _full(shape):
    return pl.BlockSpec(shape, lambda i: (0,) * len(shape))


def _mlp(emb, x_cont, gc, bc, W1, b1, g1, bt1, W2, b2, g2, bt2, W3, b3):
    W1e, W1c = W1[:_F * _D, :], W1[_F * _D:, :]
    r = lambda a: a.reshape(1, -1)

    a1, sq1 = pl.pallas_call(
        _k1_body,
        grid=(_NBLK,),
        in_specs=[
            _full((_B, _NC)),
            pl.BlockSpec((_BB, _F * _D), lambda i: (i, 0)),
            _full((_F * _D, _H1)),
            _full((_NC, _H1)),
            _full((1, _H1)),
            _full((1, _NC)),
            _full((1, _NC)),
        ],
        out_specs=[
            pl.BlockSpec((_BB, _H1), lambda i: (i, 0)),
            _full((2, _H1)),
        ],
        out_shape=[
            jax.ShapeDtypeStruct((_B, _H1), jnp.float32),
            jax.ShapeDtypeStruct((2, _H1), jnp.float32),
        ],
        scratch_shapes=[
            pltpu.VMEM((2, _H1), jnp.float32),
            pltpu.VMEM((2, _NC), jnp.float32),
        ],
    )(x_cont, emb, W1e, W1c, r(b1), r(gc), r(bc))

    a2, sq2 = pl.pallas_call(
        _k2_body,
        grid=(_NBLK,),
        in_specs=[
            pl.BlockSpec((_BB, _H1), lambda i: (i, 0)),
            _full((2, _H1)),
            _full((1, _H1)),
            _full((1, _H1)),
            _full((_H1, _H2)),
            _full((1, _H2)),
        ],
        out_specs=[
            pl.BlockSpec((_BB, _H2), lambda i: (i, 0)),
            _full((2, _H2)),
        ],
        out_shape=[
            jax.ShapeDtypeStruct((_B, _H2), jnp.float32),
            jax.ShapeDtypeStruct((2, _H2), jnp.float32),
        ],
        scratch_shapes=[pltpu.VMEM((2, _H2), jnp.float32)],
    )(a1, sq1, r(g1), r(bt1), W2, r(b2))

    out = pl.pallas_call(
        _k3_body,
        grid=(_NBLK,),
        in_specs=[
            pl.BlockSpec((_BB, _H2), lambda i: (i, 0)),
            _full((2, _H2)),
            _full((1, _H2)),
            _full((1, _H2)),
            _full((_H2, 1)),
            _full((1, 1)),
        ],
        out_specs=pl.BlockSpec((_BB, 1), lambda i: (i, 0)),
        out_shape=jax.ShapeDtypeStruct((_B, 1), jnp.float32),
    )(a2, sq2, r(g2), r(bt2), W3, r(b3))
    return out


def kernel(x_cat, x_cont, tables, gc, bc, W1, b1, g1, bt1, W2, b2, g2, bt2, W3, b3):
    tbl_flat = tables
    offsets = (jnp.arange(_F, dtype=jnp.int32) * _V)[None, :]
    idx = (x_cat.astype(jnp.int32) + offsets).reshape(-1)
    emb = _sc_gather(tbl_flat, idx).reshape(_B, _F * _D)
    return _mlp(emb, x_cont, gc, bc, W1, b1, g1, bt1, W2, b2, g2, bt2, W3, b3)


# speedup vs baseline: 1.8806x; 1.8806x over previous
import jax
import jax.numpy as jnp
from jax import lax
from jax.experimental import pallas as pl
from jax.experimental.pallas import tpu as pltpu
from jax.experimental.pallas import tpu_sc as plsc

_B = 16384
_F = 26
_V = 100000
_D = 16
_NC = 13
_H1 = 512
_H2 = 256
_EPS = 1e-5

_NW = 32
_BT = _B // _NW

_BB = 2048
_NBLK = _B // _BB


def _gather_body(tbl_hbm, xcat_hbm, out_hbm, idx_v, col_v, sem):
    c = lax.axis_index("c")
    s = lax.axis_index("s")
    wid = s * 2 + c
    b0 = wid * _BT

    def fbody(f, carry):
        pltpu.sync_copy(xcat_hbm.at[f, pl.ds(b0, _BT)], idx_v)
        cps = [
            pltpu.async_copy(tbl_hbm.at[f * _D + d].at[idx_v], col_v.at[d], sem)
            for d in range(_D)
        ]
        for cp in cps:
            cp.wait()
        pltpu.sync_copy(col_v, out_hbm.at[pl.ds(f * _D, _D), pl.ds(b0, _BT)])
        return carry

    lax.fori_loop(0, _F, fbody, 0)


def _sc_gather(tbl_t, x_cat_t):
    mesh = plsc.VectorSubcoreMesh(core_axis_name="c", subcore_axis_name="s")
    f = pl.kernel(
        _gather_body,
        out_type=jax.ShapeDtypeStruct((_F * _D, _B), jnp.float32),
        mesh=mesh,
        scratch_types=[
            pltpu.VMEM((_BT,), jnp.int32),
            pltpu.VMEM((_D, _BT), jnp.float32),
            pltpu.SemaphoreType.DMA,
        ],
        compiler_params=pltpu.CompilerParams(use_tc_tiling_on_sc=False),
    )
    return f(tbl_t, x_cat_t)


def _dot0(a, b):
    return lax.dot_general(a, b, (((0,), (0,)), ((), ())),
                           preferred_element_type=jnp.float32)


def _k1_body(xc_ref, emb_ref, w1e_ref, w1c_ref, b1_ref, gc_ref, bc_ref,
             a1_ref, sq_ref, acc_ref, xst_ref):
    pid = pl.program_id(0)

    @pl.when(pid == 0)
    def _():
        xc = xc_ref[...]
        m = jnp.mean(xc, axis=1)
        v = jnp.mean(xc * xc, axis=1) - m * m
        sc = gc_ref[:, 0] * lax.rsqrt(v + _EPS)
        xst_ref[:, 0] = sc
        xst_ref[:, 1] = bc_ref[:, 0] - m * sc
        acc_ref[...] = jnp.zeros_like(acc_ref)

    xcn = (xc_ref[:, pl.ds(pid * _BB, _BB)] * xst_ref[:, 0][:, None]
           + xst_ref[:, 1][:, None])
    z = _dot0(w1e_ref[...], emb_ref[...]) + _dot0(w1c_ref[...], xcn) + b1_ref[...]
    a1 = jnp.maximum(z, 0.0)
    a1_ref[...] = a1
    acc_ref[0, :] += jnp.sum(a1, axis=1)
    acc_ref[1, :] += jnp.sum(a1 * a1, axis=1)

    @pl.when(pid == _NBLK - 1)
    def _():
        sq_ref[...] = acc_ref[...]


def _k2_body(a1_ref, sq1_ref, g1_ref, bt1_ref, w2_ref, b2_ref,
             a2_ref, sq2_ref, acc_ref):
    pid = pl.program_id(0)

    @pl.when(pid == 0)
    def _():
        acc_ref[...] = jnp.zeros_like(acc_ref)

    m = sq1_ref[0, :] * (1.0 / _B)
    v = sq1_ref[1, :] * (1.0 / _B) - m * m
    alpha = g1_ref[:, 0] * lax.rsqrt(v + _EPS)
    beta = bt1_ref[:, 0] - m * alpha
    h1 = a1_ref[...] * alpha[:, None] + beta[:, None]
    z = _dot0(w2_ref[...], h1) + b2_ref[...]
    a2 = jnp.maximum(z, 0.0)
    a2_ref[...] = a2
    acc_ref[0, :] += jnp.sum(a2, axis=1)
    acc_ref[1, :] += jnp.sum(a2 * a2, axis=1)

    @pl.when(pid == _NBLK - 1)
    def _():
        sq2_ref[...] = acc_ref[...]


def _k3_body(a2_ref, sq2_ref, g2_ref, bt2_ref, w3_ref, b3_ref, out_ref):
    m = sq2_ref[0, :] * (1.0 / _B)
    v = sq2_ref[1, :] * (1.0 / _B) - m * m
    alpha = g2_ref[:, 0] * lax.rsqrt(v + _EPS)
    beta = bt2_ref[:, 0] - m * alpha
    h2 = a2_ref[...] * alpha[:, None] + beta[:, None]
    out_ref[...] = _dot0(w3_ref[...], h2) + b3_ref[...]


def _full(shape):
    return pl.BlockSpec(shape, lambda i: (0,) * len(shape))


def _mlp_t(emb_t, xc_t, gc, bc, W1, b1, g1, bt1, W2, b2, g2, bt2, W3, b3):
    W1e, W1c = W1[:_F * _D, :], W1[_F * _D:, :]
    col = lambda a: a.reshape(-1, 1)

    a1, sq1 = pl.pallas_call(
        _k1_body,
        grid=(_NBLK,),
        in_specs=[
            _full((_NC, _B)),
            pl.BlockSpec((_F * _D, _BB), lambda i: (0, i)),
            _full((_F * _D, _H1)),
            _full((_NC, _H1)),
            _full((_H1, 1)),
            _full((_NC, 1)),
            _full((_NC, 1)),
        ],
        out_specs=[
            pl.BlockSpec((_H1, _BB), lambda i: (0, i)),
            _full((2, _H1)),
        ],
        out_shape=[
            jax.ShapeDtypeStruct((_H1, _B), jnp.float32),
            jax.ShapeDtypeStruct((2, _H1), jnp.float32),
        ],
        scratch_shapes=[
            pltpu.VMEM((2, _H1), jnp.float32),
            pltpu.VMEM((_NC, 2), jnp.float32),
        ],
    )(xc_t, emb_t, W1e, W1c, col(b1), col(gc), col(bc))

    a2, sq2 = pl.pallas_call(
        _k2_body,
        grid=(_NBLK,),
        in_specs=[
            pl.BlockSpec((_H1, _BB), lambda i: (0, i)),
            _full((2, _H1)),
            _full((_H1, 1)),
            _full((_H1, 1)),
            _full((_H1, _H2)),
            _full((_H2, 1)),
        ],
        out_specs=[
            pl.BlockSpec((_H2, _BB), lambda i: (0, i)),
            _full((2, _H2)),
        ],
        out_shape=[
            jax.ShapeDtypeStruct((_H2, _B), jnp.float32),
            jax.ShapeDtypeStruct((2, _H2), jnp.float32),
        ],
        scratch_shapes=[pltpu.VMEM((2, _H2), jnp.float32)],
    )(a1, sq1, col(g1), col(bt1), W2, col(b2))

    out_t = pl.pallas_call(
        _k3_body,
        grid=(_NBLK,),
        in_specs=[
            pl.BlockSpec((_H2, _BB), lambda i: (0, i)),
            _full((2, _H2)),
            _full((_H2, 1)),
            _full((_H2, 1)),
            _full((_H2, 1)),
            _full((1, 1)),
        ],
        out_specs=pl.BlockSpec((1, _BB), lambda i: (0, i)),
        out_shape=jax.ShapeDtypeStruct((1, _B), jnp.float32),
    )(a2, sq2, col(g2), col(bt2), W3, b3.reshape(1, 1))
    return out_t.reshape(_B, 1)


def kernel(x_cat, x_cont, tables, gc, bc, W1, b1, g1, bt1, W2, b2, g2, bt2, W3, b3):
    tbl_t = jnp.transpose(tables, (0, 2, 1)).reshape(_F * _D, _V)
    x_cat_t = x_cat.astype(jnp.int32).T
    emb_t = _sc_gather(tbl_t, x_cat_t)
    xc_t = x_cont.T
    return _mlp_t(emb_t, xc_t, gc, bc, W1, b1, g1, bt1, W2, b2, g2, bt2, W3, b3)


# speedup vs baseline: 1.9934x; 1.0600x over previous
import jax
import jax.numpy as jnp
from jax import lax
from jax.experimental import pallas as pl
from jax.experimental.pallas import tpu as pltpu
from jax.experimental.pallas import tpu_sc as plsc

_B = 16384
_F = 26
_V = 100000
_D = 16
_NC = 13
_H1 = 512
_H2 = 256
_EPS = 1e-5

_NW = 32
_BT = _B // _NW

_BB = 2048
_NBLK = _B // _BB


def _gather_body(tbl_hbm, xcat_hbm, out_hbm, idx_v, col_v,
                 sem_g0, sem_g1, sem_o):
    c = lax.axis_index("c")
    s = lax.axis_index("s")
    wid = s * 2 + c
    b0 = wid * _BT

    def fire(f, parity_buf, sem):
        for d in range(_D):
            pltpu.async_copy(
                tbl_hbm.at[f * _D + d].at[idx_v.at[f]], col_v.at[parity_buf, d],
                sem)

    pltpu.sync_copy(xcat_hbm.at[:, pl.ds(b0, _BT)], idx_v)
    fire(0, 0, sem_g0)

    def fbody(f, carry):
        p = lax.rem(f, 2)
        pn = lax.rem(f + 1, 2)

        @pl.when(f >= 1)
        def _():
            pltpu.make_async_copy(
                col_v.at[pn],
                out_hbm.at[pl.ds((f - 1) * _D, _D), pl.ds(b0, _BT)],
                sem_o).wait()

        @pl.when(jnp.logical_and(f + 1 < _F, pn == 1))
        def _():
            fire(f + 1, 1, sem_g1)

        @pl.when(jnp.logical_and(f + 1 < _F, pn == 0))
        def _():
            fire(f + 1, 0, sem_g0)

        @pl.when(p == 0)
        def _():
            pltpu.make_async_copy(
                out_hbm.at[pl.ds(0, _D), pl.ds(0, _BT)], col_v.at[0],
                sem_g0).wait()

        @pl.when(p == 1)
        def _():
            pltpu.make_async_copy(
                out_hbm.at[pl.ds(0, _D), pl.ds(0, _BT)], col_v.at[1],
                sem_g1).wait()

        pltpu.async_copy(
            col_v.at[p], out_hbm.at[pl.ds(f * _D, _D), pl.ds(b0, _BT)], sem_o)
        return carry

    lax.fori_loop(0, _F, fbody, 0)
    pltpu.make_async_copy(
        col_v.at[1], out_hbm.at[pl.ds((_F - 1) * _D, _D), pl.ds(b0, _BT)],
        sem_o).wait()


def _sc_gather(tbl_t, x_cat_t):
    mesh = plsc.VectorSubcoreMesh(core_axis_name="c", subcore_axis_name="s")
    f = pl.kernel(
        _gather_body,
        out_type=jax.ShapeDtypeStruct((_F * _D, _B), jnp.float32),
        mesh=mesh,
        scratch_types=[
            pltpu.VMEM((_F, _BT), jnp.int32),
            pltpu.VMEM((2, _D, _BT), jnp.float32),
            pltpu.SemaphoreType.DMA,
            pltpu.SemaphoreType.DMA,
            pltpu.SemaphoreType.DMA,
        ],
        compiler_params=pltpu.CompilerParams(use_tc_tiling_on_sc=False),
    )
    return f(tbl_t, x_cat_t)


def _dot0(a, b):
    return lax.dot_general(a, b, (((0,), (0,)), ((), ())),
                           preferred_element_type=jnp.float32)


def _k1_body(xc_ref, emb_ref, w1e_ref, w1c_ref, b1_ref, gc_ref, bc_ref,
             a1_ref, sq_ref, acc_ref, xst_ref):
    pid = pl.program_id(0)

    @pl.when(pid == 0)
    def _():
        xc = xc_ref[...]
        m = jnp.mean(xc, axis=1)
        v = jnp.mean(xc * xc, axis=1) - m * m
        sc = gc_ref[:, 0] * lax.rsqrt(v + _EPS)
        xst_ref[:, 0] = sc
        xst_ref[:, 1] = bc_ref[:, 0] - m * sc
        acc_ref[...] = jnp.zeros_like(acc_ref)

    xcn = (xc_ref[:, pl.ds(pid * _BB, _BB)] * xst_ref[:, 0][:, None]
           + xst_ref[:, 1][:, None])
    z = _dot0(w1e_ref[...], emb_ref[...]) + _dot0(w1c_ref[...], xcn) + b1_ref[...]
    a1 = jnp.maximum(z, 0.0)
    a1_ref[...] = a1
    acc_ref[0, :] += jnp.sum(a1, axis=1)
    acc_ref[1, :] += jnp.sum(a1 * a1, axis=1)

    @pl.when(pid == _NBLK - 1)
    def _():
        sq_ref[...] = acc_ref[...]


def _k2_body(a1_ref, sq1_ref, g1_ref, bt1_ref, w2_ref, b2_ref,
             a2_ref, sq2_ref, acc_ref):
    pid = pl.program_id(0)

    @pl.when(pid == 0)
    def _():
        acc_ref[...] = jnp.zeros_like(acc_ref)

    m = sq1_ref[0, :] * (1.0 / _B)
    v = sq1_ref[1, :] * (1.0 / _B) - m * m
    alpha = g1_ref[:, 0] * lax.rsqrt(v + _EPS)
    beta = bt1_ref[:, 0] - m * alpha
    h1 = a1_ref[...] * alpha[:, None] + beta[:, None]
    z = _dot0(w2_ref[...], h1) + b2_ref[...]
    a2 = jnp.maximum(z, 0.0)
    a2_ref[...] = a2
    acc_ref[0, :] += jnp.sum(a2, axis=1)
    acc_ref[1, :] += jnp.sum(a2 * a2, axis=1)

    @pl.when(pid == _NBLK - 1)
    def _():
        sq2_ref[...] = acc_ref[...]


def _k3_body(a2_ref, sq2_ref, g2_ref, bt2_ref, w3_ref, b3_ref, out_ref):
    m = sq2_ref[0, :] * (1.0 / _B)
    v = sq2_ref[1, :] * (1.0 / _B) - m * m
    alpha = g2_ref[:, 0] * lax.rsqrt(v + _EPS)
    beta = bt2_ref[:, 0] - m * alpha
    h2 = a2_ref[...] * alpha[:, None] + beta[:, None]
    out_ref[...] = _dot0(w3_ref[...], h2) + b3_ref[...]


def _full(shape):
    return pl.BlockSpec(shape, lambda i: (0,) * len(shape))


def _mlp_t(emb_t, xc_t, gc, bc, W1, b1, g1, bt1, W2, b2, g2, bt2, W3, b3):
    W1e, W1c = W1[:_F * _D, :], W1[_F * _D:, :]
    col = lambda a: a.reshape(-1, 1)

    a1, sq1 = pl.pallas_call(
        _k1_body,
        grid=(_NBLK,),
        in_specs=[
            _full((_NC, _B)),
            pl.BlockSpec((_F * _D, _BB), lambda i: (0, i)),
            _full((_F * _D, _H1)),
            _full((_NC, _H1)),
            _full((_H1, 1)),
            _full((_NC, 1)),
            _full((_NC, 1)),
        ],
        out_specs=[
            pl.BlockSpec((_H1, _BB), lambda i: (0, i)),
            _full((2, _H1)),
        ],
        out_shape=[
            jax.ShapeDtypeStruct((_H1, _B), jnp.float32),
            jax.ShapeDtypeStruct((2, _H1), jnp.float32),
        ],
        scratch_shapes=[
            pltpu.VMEM((2, _H1), jnp.float32),
            pltpu.VMEM((_NC, 2), jnp.float32),
        ],
    )(xc_t, emb_t, W1e, W1c, col(b1), col(gc), col(bc))

    a2, sq2 = pl.pallas_call(
        _k2_body,
        grid=(_NBLK,),
        in_specs=[
            pl.BlockSpec((_H1, _BB), lambda i: (0, i)),
            _full((2, _H1)),
            _full((_H1, 1)),
            _full((_H1, 1)),
            _full((_H1, _H2)),
            _full((_H2, 1)),
        ],
        out_specs=[
            pl.BlockSpec((_H2, _BB), lambda i: (0, i)),
            _full((2, _H2)),
        ],
        out_shape=[
            jax.ShapeDtypeStruct((_H2, _B), jnp.float32),
            jax.ShapeDtypeStruct((2, _H2), jnp.float32),
        ],
        scratch_shapes=[pltpu.VMEM((2, _H2), jnp.float32)],
    )(a1, sq1, col(g1), col(bt1), W2, col(b2))

    out_t = pl.pallas_call(
        _k3_body,
        grid=(_NBLK,),
        in_specs=[
            pl.BlockSpec((_H2, _BB), lambda i: (0, i)),
            _full((2, _H2)),
            _full((_H2, 1)),
            _full((_H2, 1)),
            _full((_H2, 1)),
            _full((1, 1)),
        ],
        out_specs=pl.BlockSpec((1, _BB), lambda i: (0, i)),
        out_shape=jax.ShapeDtypeStruct((1, _B), jnp.float32),
    )(a2, sq2, col(g2), col(bt2), W3, b3.reshape(1, 1))
    return out_t.reshape(_B, 1)


def kernel(x_cat, x_cont, tables, gc, bc, W1, b1, g1, bt1, W2, b2, g2, bt2, W3, b3):
    tbl_t = jnp.transpose(tables, (0, 2, 1)).reshape(_F * _D, _V)
    x_cat_t = x_cat.astype(jnp.int32).T
    emb_t = _sc_gather(tbl_t, x_cat_t)
    xc_t = x_cont.T
    return _mlp_t(emb_t, xc_t, gc, bc, W1, b1, g1, bt1, W2, b2, g2, bt2, W3, b3)
